# Initial kernel scaffold; baseline (speedup 1.0000x reference)
import jax
import jax.numpy as jnp
from jax.experimental import pallas as pl


def kernel(in_feat, in_map, seg_ids):
    raise NotImplementedError("write your pallas kernel here")


# speedup vs baseline: 4.6215x; 4.6215x over previous
import functools

import jax
import jax.numpy as jnp
from jax import lax
from jax.experimental import pallas as pl
from jax.experimental.pallas import tpu as pltpu
from jax.experimental.pallas import tpu_sc as plsc

N_IN = 10000
N_OUT = 10000
D = 128
L = 16
NC = 2
NS = 16
NW = NC * NS
SEG_PER_W = 320
LAST_SEG = N_OUT - SEG_PER_W * (NW - 1)
C = 128

_i32 = jnp.int32

_mesh = plsc.VectorSubcoreMesh(core_axis_name="c", subcore_axis_name="s")


@functools.partial(
    pl.kernel,
    out_type=jax.ShapeDtypeStruct((N_OUT * D,), jnp.float32),
    mesh=_mesh,
    scratch_types=[
        pltpu.VMEM((C,), jnp.int32),
        pltpu.VMEM((C, D), jnp.float32),
        pltpu.VMEM((SEG_PER_W * D,), jnp.float32),
        pltpu.VMEM((C + L,), jnp.int32),
        pltpu.VMEM((48,), jnp.int32),
        pltpu.SemaphoreType.DMA,
    ],
)
def _sc_pool(feat_hbm, map_hbm, seg_hbm, bounds_hbm, out_hbm,
             idx_v, rows_v, out_local, seg_s, bounds_s, sem):
    cid = lax.axis_index("c")
    sid = lax.axis_index("s")
    wid = sid * _i32(NC) + cid

    pltpu.sync_copy(bounds_hbm, bounds_s)
    bvec = bounds_s[pl.ds(wid, L)]
    start = bvec[0]
    end = bvec[1]
    lo = wid * _i32(SEG_PER_W)

    zeros16 = jnp.zeros((L,), jnp.float32)

    def zero_body(i, carry):
        out_local[pl.ds(i * _i32(L), L)] = zeros16
        return carry

    lax.fori_loop(_i32(0), _i32(SEG_PER_W * D // L), zero_body, _i32(0))

    base = (start // _i32(8)) * _i32(8)
    nchunks = (end - base + _i32(C - 1)) // _i32(C)

    def chunk_body(k, carry):
        p = pl.multiple_of(base + k * _i32(C), 8)
        pltpu.sync_copy(map_hbm.at[pl.ds(p, C)], idx_v)
        pltpu.sync_copy(seg_hbm.at[pl.ds(p, C)], seg_s.at[pl.ds(0, C)])
        pltpu.async_copy(feat_hbm.at[idx_v], rows_v, sem).wait()
        j0 = jnp.maximum(start - p, _i32(0))
        j1 = jnp.minimum(end - p, _i32(C))

        def pair_body(j, pc):
            cur = pc[0]
            acc = pc[1:]
            s = seg_s[pl.ds(j, L)][0]
            changed = s != cur

            @pl.when(jnp.logical_and(changed, cur >= _i32(0)))
            def _flush():
                ob = (cur - lo) * _i32(D)
                for c in range(D // L):
                    out_local[pl.ds(ob + _i32(c * L), L)] = acc[c]

            new_acc = tuple(
                jnp.where(changed,
                          rows_v[j, pl.ds(c * L, L)],
                          jnp.maximum(acc[c], rows_v[j, pl.ds(c * L, L)]))
                for c in range(D // L))
            return (s, *new_acc)

        return lax.fori_loop(j0, j1, pair_body, carry)

    init = (jnp.int32(-1),) + tuple(
        jnp.full((L,), -jnp.inf, jnp.float32) for _ in range(D // L))
    final = lax.fori_loop(_i32(0), nchunks, chunk_body, init)
    cur = final[0]
    acc = final[1:]

    @pl.when(cur >= _i32(0))
    def _final_flush():
        ob = (cur - lo) * _i32(D)
        for c in range(D // L):
            out_local[pl.ds(ob + _i32(c * L), L)] = acc[c]

    @pl.when(wid < _i32(NW - 1))
    def _write_full():
        pltpu.sync_copy(out_local, out_hbm.at[pl.ds(lo * _i32(D), SEG_PER_W * D)])

    @pl.when(wid == _i32(NW - 1))
    def _write_last():
        pltpu.sync_copy(out_local.at[pl.ds(0, LAST_SEG * D)],
                        out_hbm.at[pl.ds(lo * _i32(D), LAST_SEG * D)])


def kernel(in_feat, in_map, seg_ids):
    map32 = in_map.astype(jnp.int32)
    seg32 = seg_ids.astype(jnp.int32)
    targets = jnp.arange(NW + 1, dtype=jnp.int32) * SEG_PER_W
    bounds = jnp.searchsorted(seg32, targets, side="left").astype(jnp.int32)
    bounds = jnp.pad(bounds, (0, 48 - (NW + 1)))
    map_p = jnp.concatenate([map32, jnp.zeros((C,), jnp.int32)])
    seg_p = jnp.concatenate([seg32, jnp.full((C,), N_OUT, jnp.int32)])
    out = _sc_pool(in_feat.astype(jnp.float32), map_p, seg_p, bounds)
    return out.reshape(N_OUT, D)
